# Initial kernel scaffold; baseline (speedup 1.0000x reference)
import jax
import jax.numpy as jnp
from jax.experimental import pallas as pl


def kernel(x):
    raise NotImplementedError("write your pallas kernel here")


# speedup vs baseline: 1.5378x; 1.5378x over previous
import numpy as np
import jax
import jax.numpy as jnp
from jax import lax
from jax.experimental import pallas as pl
from jax.experimental.pallas import tpu as pltpu
from jax.experimental.pallas import tpu_sc as plsc

_B, _C, _L = 16, 192, 4096
_S = 64
_ROWS = _B * _C
_NC, _NS = 2, 16
_NW = _NC * _NS
_RPW = _ROWS // _NW
_LANES = 16


def _morton_idx(l: int) -> np.ndarray:
    s = int(np.sqrt(l))
    ij = np.arange(l, dtype=np.int64)
    i = np.zeros(l, dtype=np.int64)
    j = np.zeros(l, dtype=np.int64)
    for t in range(int(l).bit_length() // 2 + 1):
        i += ((ij >> (2 * t + 1)) & 1) << t
        j += ((ij >> (2 * t)) & 1) << t
    return (i * s + j).astype(np.int32)


_IDX_NP = _morton_idx(_L)


def _sc_body(x_hbm, idx_hbm, out_hbm, idx_v, in_v, out_v):
    wid = lax.axis_index("s") * _NC + lax.axis_index("c")
    base = wid * _RPW
    pltpu.sync_copy(idx_hbm, idx_v)

    @pl.loop(0, _RPW)
    def _row(r):
        row = base + r
        pltpu.sync_copy(x_hbm.at[row], in_v)

        @pl.loop(0, _L // _LANES)
        def _blk(k):
            v = in_v[pl.ds(k * _LANES, _LANES)]
            iv = idx_v[pl.ds(k * _LANES, _LANES)]
            plsc.store_scatter(out_v, [iv], v)

        pltpu.sync_copy(out_v, out_hbm.at[row])


def kernel(x):
    xf = x.reshape(_ROWS, _L)
    idx = jnp.asarray(_IDX_NP, dtype=jnp.int32)
    mesh = plsc.VectorSubcoreMesh(core_axis_name="c", subcore_axis_name="s")
    out = pl.kernel(
        _sc_body,
        out_type=jax.ShapeDtypeStruct((_ROWS, _L), jnp.float32),
        mesh=mesh,
        scratch_types=[
            pltpu.VMEM((_L,), jnp.int32),
            pltpu.VMEM((_L,), jnp.float32),
            pltpu.VMEM((_L,), jnp.float32),
        ],
        compiler_params=pltpu.CompilerParams(needs_layout_passes=False),
    )(xf, idx)
    return out.reshape(_B, _C, _S, _S)


# speedup vs baseline: 1.8167x; 1.1814x over previous
import numpy as np
import jax
import jax.numpy as jnp
from jax import lax
from jax.experimental import pallas as pl
from jax.experimental.pallas import tpu as pltpu
from jax.experimental.pallas import tpu_sc as plsc

_B, _C, _L = 16, 192, 4096
_S = 64
_ROWS = _B * _C
_NC, _NS = 2, 16
_NW = _NC * _NS
_RPW = _ROWS // _NW
_LANES = 16
_G = 4
_GL = _G * _L
_NG = _RPW // _G


def _morton_idx(l: int) -> np.ndarray:
    s = int(np.sqrt(l))
    ij = np.arange(l, dtype=np.int64)
    i = np.zeros(l, dtype=np.int64)
    j = np.zeros(l, dtype=np.int64)
    for t in range(int(l).bit_length() // 2 + 1):
        i += ((ij >> (2 * t + 1)) & 1) << t
        j += ((ij >> (2 * t)) & 1) << t
    return (i * s + j).astype(np.int32)


_IDX_NP = _morton_idx(_L)


def _sc_body(x_hbm, idx_hbm, out_hbm, idx_v, in_v, out_v, in_sem, out_sem):
    wid = lax.axis_index("s") * _NC + lax.axis_index("c")
    base = wid * _RPW * _L
    pltpu.sync_copy(idx_hbm, idx_v)

    def load(g, b):
        pltpu.async_copy(x_hbm.at[pl.ds(base + g * _GL, _GL)],
                         in_v.at[pl.ds(b * _GL, _GL)], in_sem.at[b])

    def wait_in(b):
        pltpu.make_async_copy(x_hbm.at[pl.ds(0, _GL)],
                              in_v.at[pl.ds(b * _GL, _GL)],
                              in_sem.at[b]).wait()

    def store(g, b):
        pltpu.async_copy(out_v.at[pl.ds(b * _GL, _GL)],
                         out_hbm.at[pl.ds(base + g * _GL, _GL)], out_sem.at[b])

    def wait_out(b):
        pltpu.make_async_copy(out_v.at[pl.ds(b * _GL, _GL)],
                              out_hbm.at[pl.ds(0, _GL)], out_sem.at[b]).wait()

    load(0, 0)

    @pl.loop(0, _NG, step=2)
    def _grp(g0):
        for b in range(2):
            g = g0 + b

            @pl.when(g + 1 < _NG)
            def _():
                load(g + 1, 1 - b)

            wait_in(b)

            @pl.when(g >= 2)
            def _():
                wait_out(b)

            for r in range(_G):
                offs = (b * _G + r) * _L

                @pl.loop(0, _L // _LANES, unroll=8)
                def _blk(k):
                    v = in_v[pl.ds(offs + k * _LANES, _LANES)]
                    iv = idx_v[pl.ds(k * _LANES, _LANES)] + offs
                    plsc.store_scatter(out_v, [iv], v)

            store(g, b)

    wait_out(0)
    wait_out(1)


def kernel(x):
    xf = x.reshape(_ROWS * _L)
    idx = jnp.asarray(_IDX_NP, dtype=jnp.int32)
    mesh = plsc.VectorSubcoreMesh(core_axis_name="c", subcore_axis_name="s")
    out = pl.kernel(
        _sc_body,
        out_type=jax.ShapeDtypeStruct((_ROWS * _L,), jnp.float32),
        mesh=mesh,
        scratch_types=[
            pltpu.VMEM((_L,), jnp.int32),
            pltpu.VMEM((2 * _GL,), jnp.float32),
            pltpu.VMEM((2 * _GL,), jnp.float32),
            pltpu.SemaphoreType.DMA((2,)),
            pltpu.SemaphoreType.DMA((2,)),
        ],
        compiler_params=pltpu.CompilerParams(needs_layout_passes=False),
    )(xf, idx)
    return out.reshape(_B, _C, _S, _S)


# speedup vs baseline: 2.7086x; 1.4909x over previous
import numpy as np
import jax
import jax.numpy as jnp
from jax import lax
from jax.experimental import pallas as pl
from jax.experimental.pallas import tpu as pltpu
from jax.experimental.pallas import tpu_sc as plsc

_B, _C, _L = 16, 192, 4096
_S = 64
_ROWS = _B * _C
_NC, _NS = 2, 16
_NW = _NC * _NS
_RPW = _ROWS // _NW
_LANES = 16
_G = 4
_GL = _G * _L
_NG = _RPW // _G


def _morton_idx(l: int) -> np.ndarray:
    s = int(np.sqrt(l))
    ij = np.arange(l, dtype=np.int64)
    i = np.zeros(l, dtype=np.int64)
    j = np.zeros(l, dtype=np.int64)
    for t in range(int(l).bit_length() // 2 + 1):
        i += ((ij >> (2 * t + 1)) & 1) << t
        j += ((ij >> (2 * t)) & 1) << t
    return (i * s + j).astype(np.int32)


_IDX_NP = (_morton_idx(_L)[None, :] +
           (np.arange(2 * _G, dtype=np.int32) * _L)[:, None]).reshape(-1)


def _sc_body(x_hbm, idx_hbm, out_hbm, idx_v, in_v, out_v, in_sem, out_sem):
    wid = lax.axis_index("s") * _NC + lax.axis_index("c")
    base = wid * _RPW * _L
    pltpu.sync_copy(idx_hbm, idx_v)

    def load(g, b):
        pltpu.async_copy(x_hbm.at[pl.ds(base + g * _GL, _GL)],
                         in_v.at[pl.ds(b * _GL, _GL)], in_sem.at[b])

    def wait_in(b):
        pltpu.make_async_copy(x_hbm.at[pl.ds(0, _GL)],
                              in_v.at[pl.ds(b * _GL, _GL)],
                              in_sem.at[b]).wait()

    def store(g, b):
        pltpu.async_copy(out_v.at[pl.ds(b * _GL, _GL)],
                         out_hbm.at[pl.ds(base + g * _GL, _GL)], out_sem.at[b])

    def wait_out(b):
        pltpu.make_async_copy(out_v.at[pl.ds(b * _GL, _GL)],
                              out_hbm.at[pl.ds(0, _GL)], out_sem.at[b]).wait()

    load(0, 0)

    @pl.loop(0, _NG, step=2)
    def _grp(g0):
        for b in range(2):
            g = g0 + b

            @pl.when(g + 1 < _NG)
            def _():
                load(g + 1, 1 - b)

            wait_in(b)

            @pl.when(g >= 2)
            def _():
                wait_out(b)

            o = b * _GL

            @plsc.parallel_loop(0, _GL // _LANES, unroll=8)
            def _blk(k):
                p = o + k * _LANES
                v = in_v[pl.ds(p, _LANES)]
                iv = idx_v[pl.ds(p, _LANES)]
                plsc.store_scatter(out_v, [iv], v)

            store(g, b)

    wait_out(0)
    wait_out(1)


def kernel(x):
    xf = x.reshape(_ROWS * _L)
    idx = jnp.asarray(_IDX_NP, dtype=jnp.int32)
    mesh = plsc.VectorSubcoreMesh(core_axis_name="c", subcore_axis_name="s")
    out = pl.kernel(
        _sc_body,
        out_type=jax.ShapeDtypeStruct((_ROWS * _L,), jnp.float32),
        mesh=mesh,
        scratch_types=[
            pltpu.VMEM((2 * _GL,), jnp.int32),
            pltpu.VMEM((2 * _GL,), jnp.float32),
            pltpu.VMEM((2 * _GL,), jnp.float32),
            pltpu.SemaphoreType.DMA((2,)),
            pltpu.SemaphoreType.DMA((2,)),
        ],
        compiler_params=pltpu.CompilerParams(needs_layout_passes=False),
    )(xf, idx)
    return out.reshape(_B, _C, _S, _S)
